# Initial kernel scaffold; baseline (speedup 1.0000x reference)
import jax
import jax.numpy as jnp
from jax.experimental import pallas as pl


def kernel(node_features, edge_index, edge_features, W1, b1, W2, b2, Wfc, bfc):
    raise NotImplementedError("write your pallas kernel here")


# speedup vs baseline: 13.3985x; 13.3985x over previous
import functools

import jax
import jax.numpy as jnp
from jax import lax
from jax.experimental import pallas as pl
from jax.experimental.pallas import tpu as pltpu
from jax.experimental.pallas import tpu_sc as plsc

NC = 2
NS = 16
NW = NC * NS
LANES = 16
CH = 80
NPAD = 10240


def _mesh():
    return plsc.VectorSubcoreMesh(core_axis_name="c", subcore_axis_name="s")


def _deg_call(E):
    rows = (E // NS) // CH
    rpt = NPAD // NS

    @functools.partial(
        pl.kernel,
        out_type=(jax.ShapeDtypeStruct((NPAD,), jnp.float32),
                  jax.ShapeDtypeStruct((NPAD,), jnp.float32)),
        mesh=_mesh(),
        scratch_types=[
            pltpu.VMEM_SHARED((NPAD,), jnp.float32),
            pltpu.VMEM((rows, 1, CH), jnp.int32),
            pltpu.VMEM((CH,), jnp.float32),
        ],
    )
    def deg_kernel(src3d, dst3d, ones_h, zeros_h, out_o, out_i,
                   deg_sh, idx_v, ones_v):
        c = lax.axis_index("c")
        s = lax.axis_index("s")
        z0 = s * rpt
        pltpu.sync_copy(zeros_h.at[pl.ds(z0, rpt)],
                        deg_sh.at[pl.ds(z0, rpt)])
        pltpu.sync_copy(ones_h, ones_v)

        @pl.when(c == 0)
        def _():
            pltpu.sync_copy(src3d.at[s], idx_v)

        @pl.when(c == 1)
        def _():
            pltpu.sync_copy(dst3d.at[s], idx_v)

        plsc.subcore_barrier()

        def step(j, carry):
            pltpu.sync_copy(ones_v, deg_sh.at[idx_v.at[j, 0]], add=True)
            return carry

        lax.fori_loop(0, rows, step, 0)
        plsc.subcore_barrier()

        @pl.when(c == 0)
        def _():
            pltpu.sync_copy(deg_sh.at[pl.ds(z0, rpt)], out_o.at[pl.ds(z0, rpt)])

        @pl.when(c == 1)
        def _():
            pltpu.sync_copy(deg_sh.at[pl.ds(z0, rpt)], out_i.at[pl.ds(z0, rpt)])

    return deg_kernel


def _agg_call(E, D):
    rows = (E // NW) // CH
    rpt = NPAD // NS

    @functools.partial(
        pl.kernel,
        out_type=jax.ShapeDtypeStruct((NC, NPAD, D), jnp.float32),
        mesh=_mesh(),
        scratch_types=[
            pltpu.VMEM_SHARED((NPAD, D), jnp.float32),
            pltpu.VMEM((rows, 1, CH), jnp.int32),
            pltpu.VMEM((rows, 1, CH), jnp.int32),
            pltpu.VMEM((CH, D), jnp.float32),
            pltpu.SemaphoreType.DMA,
        ],
    )
    def agg_kernel(table, src4d, dst4d, zeros_h, out,
                   acc_sh, si_v, di_v, rows_v, gsem):
        c = lax.axis_index("c")
        s = lax.axis_index("s")
        r0 = s * rpt
        pltpu.sync_copy(zeros_h.at[pl.ds(r0, rpt)], acc_sh.at[pl.ds(r0, rpt)])
        w = c * NS + s
        pltpu.sync_copy(src4d.at[w], si_v)
        pltpu.sync_copy(dst4d.at[w], di_v)
        plsc.subcore_barrier()

        def step(j, carry):
            pltpu.async_copy(table.at[si_v.at[j, 0]], rows_v, gsem).wait()
            pltpu.sync_copy(rows_v, acc_sh.at[di_v.at[j, 0]], add=True)
            return carry

        lax.fori_loop(0, rows, step, 0)
        plsc.subcore_barrier()

        @pl.when(c == 0)
        def _():
            pltpu.sync_copy(acc_sh.at[pl.ds(r0, rpt)], out.at[0, pl.ds(r0, rpt)])

        @pl.when(c == 1)
        def _():
            pltpu.sync_copy(acc_sh.at[pl.ds(r0, rpt)], out.at[1, pl.ds(r0, rpt)])

    return agg_kernel


def _edge_call(E, N):
    ept = E // NW
    blk = 2000
    nb = ept // blk
    nv = blk // LANES

    @functools.partial(
        pl.kernel,
        out_type=jax.ShapeDtypeStruct((E,), jnp.float32),
        mesh=_mesh(),
        scratch_types=[
            pltpu.VMEM((2 * N,), jnp.float32),
            pltpu.VMEM((blk,), jnp.int32),
            pltpu.VMEM((blk,), jnp.int32),
            pltpu.VMEM((blk,), jnp.float32),
            pltpu.VMEM((blk,), jnp.float32),
            pltpu.VMEM((blk,), jnp.float32),
            pltpu.VMEM((blk,), jnp.float32),
            pltpu.VMEM((LANES,), jnp.float32),
        ],
        compiler_params=pltpu.CompilerParams(needs_layout_passes=False),
    )
    def edge_kernel(ab, src, dst, ef0, ef1, ef2, wcb, out,
                    ab_v, s_v, d_v, e0_v, e1_v, e2_v, o_v, w_v):
        c = lax.axis_index("c")
        s = lax.axis_index("s")
        base = (c * NS + s) * ept
        pltpu.sync_copy(ab, ab_v)
        pltpu.sync_copy(wcb, w_v)
        wvec = w_v[...]
        wc0 = wvec[0]
        wc1 = wvec[1]
        wc2 = wvec[2]
        bfc = wvec[3]

        def blk_step(bi, carry):
            off = base + bi * blk
            pltpu.sync_copy(src.at[pl.ds(off, blk)], s_v)
            pltpu.sync_copy(dst.at[pl.ds(off, blk)], d_v)
            pltpu.sync_copy(ef0.at[pl.ds(off, blk)], e0_v)
            pltpu.sync_copy(ef1.at[pl.ds(off, blk)], e1_v)
            pltpu.sync_copy(ef2.at[pl.ds(off, blk)], e2_v)

            def vstep(k, carry2):
                sl = pl.ds(k * LANES, LANES)
                av = plsc.load_gather(ab_v, [s_v[sl] * 2])
                bv = plsc.load_gather(ab_v, [d_v[sl] * 2 + 1])
                x = (av + bv + e0_v[sl] * wc0 + e1_v[sl] * wc1
                     + e2_v[sl] * wc2 + bfc)
                o_v[sl] = 1.0 / (1.0 + jnp.exp(-x))
                return carry2

            lax.fori_loop(0, nv, vstep, 0)
            pltpu.sync_copy(o_v, out.at[pl.ds(off, blk)])
            return carry

        lax.fori_loop(0, nb, blk_step, 0)

    return edge_kernel


def _tc_pre(x, W1, deg_t):
    N, D = x.shape
    H = W1.shape[1]
    R = 1000

    def body(x_ref, w_ref, dg_ref, o_ref):
        ns = lax.rsqrt(jnp.maximum(dg_ref[:, 0:1], 1.0))
        o_ref[...] = jnp.dot(x_ref[...], w_ref[...],
                             preferred_element_type=jnp.float32) * ns

    return pl.pallas_call(
        body,
        grid=(N // R,),
        in_specs=[
            pl.BlockSpec((R, D), lambda i: (i, 0)),
            pl.BlockSpec((D, H), lambda i: (0, 0)),
            pl.BlockSpec((R, 2), lambda i: (i, 0)),
        ],
        out_specs=pl.BlockSpec((R, H), lambda i: (i, 0)),
        out_shape=jax.ShapeDtypeStruct((N, H), jnp.float32),
    )(x, W1, deg_t)


def _tc_mid(acc, deg_t, b1, W2):
    N = deg_t.shape[0]
    H = W2.shape[0]
    R = 1000

    def body(a_ref, dg_ref, b_ref, w_ref, o_ref):
        asum = a_ref[0] + a_ref[1]
        nd = lax.rsqrt(jnp.maximum(dg_ref[:, 1:2], 1.0))
        h1 = jnp.maximum(asum * nd + b_ref[...], 0.0)
        ns = lax.rsqrt(jnp.maximum(dg_ref[:, 0:1], 1.0))
        o_ref[...] = jnp.dot(h1, w_ref[...],
                             preferred_element_type=jnp.float32) * ns

    return pl.pallas_call(
        body,
        grid=(N // R,),
        in_specs=[
            pl.BlockSpec((2, R, H), lambda i: (0, i, 0)),
            pl.BlockSpec((R, 2), lambda i: (i, 0)),
            pl.BlockSpec((1, H), lambda i: (0, 0)),
            pl.BlockSpec((H, H), lambda i: (0, 0)),
        ],
        out_specs=pl.BlockSpec((R, H), lambda i: (i, 0)),
        out_shape=jax.ShapeDtypeStruct((N, H), jnp.float32),
    )(acc, deg_t, b1, W2)


def _tc_post(acc, deg_t, b2, Wab):
    N = deg_t.shape[0]
    H = Wab.shape[0]
    R = 1000

    def body(a_ref, dg_ref, b_ref, w_ref, o_ref):
        asum = a_ref[0] + a_ref[1]
        nd = lax.rsqrt(jnp.maximum(dg_ref[:, 1:2], 1.0))
        h2 = asum * nd + b_ref[...]
        o_ref[...] = jnp.dot(h2, w_ref[...],
                             preferred_element_type=jnp.float32)

    return pl.pallas_call(
        body,
        grid=(N // R,),
        in_specs=[
            pl.BlockSpec((2, R, H), lambda i: (0, i, 0)),
            pl.BlockSpec((R, 2), lambda i: (i, 0)),
            pl.BlockSpec((1, H), lambda i: (0, 0)),
            pl.BlockSpec((H, 2), lambda i: (0, 0)),
        ],
        out_specs=pl.BlockSpec((R, 2), lambda i: (i, 0)),
        out_shape=jax.ShapeDtypeStruct((N, 2), jnp.float32),
    )(acc, deg_t, b2, Wab)


def kernel(node_features, edge_index, edge_features, W1, b1, W2, b2, Wfc, bfc):
    N, D = node_features.shape
    E = edge_index.shape[1]
    H = W1.shape[1]

    src = edge_index[0]
    dst = edge_index[1]
    src3d = src.reshape(NS, (E // NS) // CH, 1, CH)
    dst3d = dst.reshape(NS, (E // NS) // CH, 1, CH)
    src4d = src.reshape(NW, (E // NW) // CH, 1, CH)
    dst4d = dst.reshape(NW, (E // NW) // CH, 1, CH)
    eft = edge_features.T
    ef0 = eft[0]
    ef1 = eft[1]
    ef2 = eft[2]

    zeros_nd = jnp.zeros((NPAD, H), jnp.float32)
    zeros_p = jnp.zeros((NPAD,), jnp.float32)
    ones_c = jnp.ones((CH,), jnp.float32)

    deg_o, deg_i = _deg_call(E)(src3d, dst3d, ones_c, zeros_p)
    deg_t = jnp.stack([deg_o[:N], deg_i[:N]], axis=1)

    xs1 = _tc_pre(node_features, W1, deg_t)
    acc1 = _agg_call(E, H)(xs1, src4d, dst4d, zeros_nd)
    xs2 = _tc_mid(acc1, deg_t, b1.reshape(1, H), W2)
    acc2 = _agg_call(E, H)(xs2, src4d, dst4d, zeros_nd)

    Wab = jnp.concatenate([Wfc[:H], Wfc[H:2 * H]], axis=1)
    ab = _tc_post(acc2, deg_t, b2.reshape(1, H), Wab)
    ab_flat = ab.reshape(2 * N)

    wcb = jnp.concatenate(
        [Wfc[2 * H:, 0], bfc, jnp.zeros((12,), jnp.float32)])
    return _edge_call(E, N)(ab_flat, src, dst, ef0, ef1, ef2, wcb)


# speedup vs baseline: 16.6394x; 1.2419x over previous
import functools

import jax
import jax.numpy as jnp
from jax import lax
from jax.experimental import pallas as pl
from jax.experimental.pallas import tpu as pltpu
from jax.experimental.pallas import tpu_sc as plsc

NC = 2
NS = 16
NW = NC * NS
LANES = 16
CH = 80
NPAD = 10240


def _mesh():
    return plsc.VectorSubcoreMesh(core_axis_name="c", subcore_axis_name="s")


def _deg_call(E):
    ept = E // NS
    rows = ept // CH
    rpt = NPAD // NS

    @functools.partial(
        pl.kernel,
        out_type=(jax.ShapeDtypeStruct((NPAD,), jnp.float32),
                  jax.ShapeDtypeStruct((NPAD,), jnp.float32)),
        mesh=_mesh(),
        scratch_types=[
            pltpu.VMEM_SHARED((NPAD,), jnp.float32),
            pltpu.VMEM((ept,), jnp.int32),
            pltpu.VMEM((CH,), jnp.float32),
        ],
    )
    def deg_kernel(src_h, dst_h, ones_h, zeros_h, out_o, out_i,
                   deg_sh, idx_v, ones_v):
        c = lax.axis_index("c")
        s = lax.axis_index("s")
        z0 = s * rpt
        pltpu.sync_copy(zeros_h.at[pl.ds(z0, rpt)],
                        deg_sh.at[pl.ds(z0, rpt)])
        pltpu.sync_copy(ones_h, ones_v)

        e0 = s * ept

        @pl.when(c == 0)
        def _():
            pltpu.sync_copy(src_h.at[pl.ds(e0, ept)], idx_v)

        @pl.when(c == 1)
        def _():
            pltpu.sync_copy(dst_h.at[pl.ds(e0, ept)], idx_v)

        plsc.subcore_barrier()

        def step(j, carry):
            o = pl.multiple_of(j * CH, 8)
            pltpu.sync_copy(ones_v, deg_sh.at[idx_v.at[pl.ds(o, CH)]],
                            add=True)
            return carry

        lax.fori_loop(0, rows, step, 0)
        plsc.subcore_barrier()

        @pl.when(c == 0)
        def _():
            pltpu.sync_copy(deg_sh.at[pl.ds(z0, rpt)], out_o.at[pl.ds(z0, rpt)])

        @pl.when(c == 1)
        def _():
            pltpu.sync_copy(deg_sh.at[pl.ds(z0, rpt)], out_i.at[pl.ds(z0, rpt)])

    return deg_kernel


def _agg_call(E, D):
    ept = E // NW
    rows = ept // CH
    rpt = NPAD // NS

    @functools.partial(
        pl.kernel,
        out_type=jax.ShapeDtypeStruct((NC, NPAD, D), jnp.float32),
        mesh=_mesh(),
        scratch_types=[
            pltpu.VMEM_SHARED((NPAD, D), jnp.float32),
            pltpu.VMEM((ept,), jnp.int32),
            pltpu.VMEM((ept,), jnp.int32),
            pltpu.VMEM((CH, D), jnp.float32),
            pltpu.VMEM((CH, D), jnp.float32),
            pltpu.SemaphoreType.DMA,
            pltpu.SemaphoreType.DMA,
            pltpu.SemaphoreType.DMA,
            pltpu.SemaphoreType.DMA,
        ],
    )
    def agg_kernel(table, src_h, dst_h, zeros_h, out,
                   acc_sh, si_v, di_v, buf0, buf1, g0, g1, s0, s1):
        c = lax.axis_index("c")
        s = lax.axis_index("s")
        r0 = s * rpt
        pltpu.sync_copy(zeros_h.at[pl.ds(r0, rpt)], acc_sh.at[pl.ds(r0, rpt)])
        w = c * NS + s
        e0 = w * ept
        pltpu.sync_copy(src_h.at[pl.ds(e0, ept)], si_v)
        pltpu.sync_copy(dst_h.at[pl.ds(e0, ept)], di_v)
        plsc.subcore_barrier()

        def sidx(j):
            return si_v.at[pl.ds(pl.multiple_of(j * CH, 8), CH)]

        def didx(j):
            return di_v.at[pl.ds(pl.multiple_of(j * CH, 8), CH)]

        pltpu.async_copy(table.at[sidx(0)], buf0, g0)

        def step(j, carry):
            def halfstep(bufa, bufb, ga, gb, sa, sb):
                pltpu.make_async_copy(table.at[sidx(j)], bufa, ga).wait()

                @pl.when(j >= 1)
                def _():
                    pltpu.make_async_copy(bufb, acc_sh.at[didx(j - 1)],
                                          sb).wait()

                @pl.when(j + 1 < rows)
                def _():
                    pltpu.async_copy(table.at[sidx(j + 1)], bufb, gb)

                pltpu.async_copy(bufa, acc_sh.at[didx(j)], sa, add=True)

            @pl.when(j % 2 == 0)
            def _():
                halfstep(buf0, buf1, g0, g1, s0, s1)

            @pl.when(j % 2 == 1)
            def _():
                halfstep(buf1, buf0, g1, g0, s1, s0)

            return carry

        lax.fori_loop(0, rows, step, 0)
        if rows % 2 == 1:
            pltpu.make_async_copy(buf0, acc_sh.at[didx(rows - 1)], s0).wait()
        else:
            pltpu.make_async_copy(buf1, acc_sh.at[didx(rows - 1)], s1).wait()
        plsc.subcore_barrier()

        @pl.when(c == 0)
        def _():
            pltpu.sync_copy(acc_sh.at[pl.ds(r0, rpt)], out.at[0, pl.ds(r0, rpt)])

        @pl.when(c == 1)
        def _():
            pltpu.sync_copy(acc_sh.at[pl.ds(r0, rpt)], out.at[1, pl.ds(r0, rpt)])

    return agg_kernel


def _edge_call(E, N):
    ept = E // NW
    blk = 2000
    nb = ept // blk
    nv = blk // LANES

    @functools.partial(
        pl.kernel,
        out_type=jax.ShapeDtypeStruct((E,), jnp.float32),
        mesh=_mesh(),
        scratch_types=[
            pltpu.VMEM((2 * N,), jnp.float32),
            pltpu.VMEM((blk,), jnp.int32),
            pltpu.VMEM((blk,), jnp.int32),
            pltpu.VMEM((blk,), jnp.float32),
            pltpu.VMEM((blk,), jnp.float32),
            pltpu.VMEM((blk,), jnp.float32),
            pltpu.VMEM((blk,), jnp.float32),
            pltpu.VMEM((LANES,), jnp.float32),
        ],
        compiler_params=pltpu.CompilerParams(needs_layout_passes=False),
    )
    def edge_kernel(ab, src, dst, ef0, ef1, ef2, wcb, out,
                    ab_v, s_v, d_v, e0_v, e1_v, e2_v, o_v, w_v):
        c = lax.axis_index("c")
        s = lax.axis_index("s")
        base = (c * NS + s) * ept
        pltpu.sync_copy(ab, ab_v)
        pltpu.sync_copy(wcb, w_v)
        wvec = w_v[...]
        wc0 = wvec[0]
        wc1 = wvec[1]
        wc2 = wvec[2]
        bfc = wvec[3]

        def blk_step(bi, carry):
            off = base + bi * blk
            pltpu.sync_copy(src.at[pl.ds(off, blk)], s_v)
            pltpu.sync_copy(dst.at[pl.ds(off, blk)], d_v)
            pltpu.sync_copy(ef0.at[pl.ds(off, blk)], e0_v)
            pltpu.sync_copy(ef1.at[pl.ds(off, blk)], e1_v)
            pltpu.sync_copy(ef2.at[pl.ds(off, blk)], e2_v)

            def vstep(k, carry2):
                sl = pl.ds(k * LANES, LANES)
                av = plsc.load_gather(ab_v, [s_v[sl] * 2])
                bv = plsc.load_gather(ab_v, [d_v[sl] * 2 + 1])
                x = (av + bv + e0_v[sl] * wc0 + e1_v[sl] * wc1
                     + e2_v[sl] * wc2 + bfc)
                o_v[sl] = 1.0 / (1.0 + jnp.exp(-x))
                return carry2

            lax.fori_loop(0, nv, vstep, 0)
            pltpu.sync_copy(o_v, out.at[pl.ds(off, blk)])
            return carry

        lax.fori_loop(0, nb, blk_step, 0)

    return edge_kernel


def _tc_pre(x, W1, deg_t):
    N, D = x.shape
    H = W1.shape[1]
    R = 1000

    def body(x_ref, w_ref, dg_ref, o_ref):
        ns = lax.rsqrt(jnp.maximum(dg_ref[:, 0:1], 1.0))
        o_ref[...] = jnp.dot(x_ref[...], w_ref[...],
                             preferred_element_type=jnp.float32) * ns

    return pl.pallas_call(
        body,
        grid=(N // R,),
        in_specs=[
            pl.BlockSpec((R, D), lambda i: (i, 0)),
            pl.BlockSpec((D, H), lambda i: (0, 0)),
            pl.BlockSpec((R, 2), lambda i: (i, 0)),
        ],
        out_specs=pl.BlockSpec((R, H), lambda i: (i, 0)),
        out_shape=jax.ShapeDtypeStruct((N, H), jnp.float32),
    )(x, W1, deg_t)


def _tc_mid(acc, deg_t, b1, W2):
    N = deg_t.shape[0]
    H = W2.shape[0]
    R = 1000

    def body(a_ref, dg_ref, b_ref, w_ref, o_ref):
        asum = a_ref[0] + a_ref[1]
        nd = lax.rsqrt(jnp.maximum(dg_ref[:, 1:2], 1.0))
        h1 = jnp.maximum(asum * nd + b_ref[...], 0.0)
        ns = lax.rsqrt(jnp.maximum(dg_ref[:, 0:1], 1.0))
        o_ref[...] = jnp.dot(h1, w_ref[...],
                             preferred_element_type=jnp.float32) * ns

    return pl.pallas_call(
        body,
        grid=(N // R,),
        in_specs=[
            pl.BlockSpec((2, R, H), lambda i: (0, i, 0)),
            pl.BlockSpec((R, 2), lambda i: (i, 0)),
            pl.BlockSpec((1, H), lambda i: (0, 0)),
            pl.BlockSpec((H, H), lambda i: (0, 0)),
        ],
        out_specs=pl.BlockSpec((R, H), lambda i: (i, 0)),
        out_shape=jax.ShapeDtypeStruct((N, H), jnp.float32),
    )(acc, deg_t, b1, W2)


def _tc_post(acc, deg_t, b2, Wab):
    N = deg_t.shape[0]
    H = Wab.shape[0]
    R = 1000

    def body(a_ref, dg_ref, b_ref, w_ref, o_ref):
        asum = a_ref[0] + a_ref[1]
        nd = lax.rsqrt(jnp.maximum(dg_ref[:, 1:2], 1.0))
        h2 = asum * nd + b_ref[...]
        o_ref[...] = jnp.dot(h2, w_ref[...],
                             preferred_element_type=jnp.float32)

    return pl.pallas_call(
        body,
        grid=(N // R,),
        in_specs=[
            pl.BlockSpec((2, R, H), lambda i: (0, i, 0)),
            pl.BlockSpec((R, 2), lambda i: (i, 0)),
            pl.BlockSpec((1, H), lambda i: (0, 0)),
            pl.BlockSpec((H, 2), lambda i: (0, 0)),
        ],
        out_specs=pl.BlockSpec((R, 2), lambda i: (i, 0)),
        out_shape=jax.ShapeDtypeStruct((N, 2), jnp.float32),
    )(acc, deg_t, b2, Wab)


def kernel(node_features, edge_index, edge_features, W1, b1, W2, b2, Wfc, bfc):
    N, D = node_features.shape
    E = edge_index.shape[1]
    H = W1.shape[1]

    src = edge_index[0]
    dst = edge_index[1]
    eft = edge_features.T
    ef0 = eft[0]
    ef1 = eft[1]
    ef2 = eft[2]

    zeros_nd = jnp.zeros((NPAD, H), jnp.float32)
    zeros_p = jnp.zeros((NPAD,), jnp.float32)
    ones_c = jnp.ones((CH,), jnp.float32)

    deg_o, deg_i = _deg_call(E)(src, dst, ones_c, zeros_p)
    deg_t = jnp.stack([deg_o[:N], deg_i[:N]], axis=1)

    xs1 = _tc_pre(node_features, W1, deg_t)
    acc1 = _agg_call(E, H)(xs1, src, dst, zeros_nd)
    xs2 = _tc_mid(acc1, deg_t, b1.reshape(1, H), W2)
    acc2 = _agg_call(E, H)(xs2, src, dst, zeros_nd)

    Wab = jnp.concatenate([Wfc[:H], Wfc[H:2 * H]], axis=1)
    ab = _tc_post(acc2, deg_t, b2.reshape(1, H), Wab)
    ab_flat = ab.reshape(2 * N)

    wcb = jnp.concatenate(
        [Wfc[2 * H:, 0], bfc, jnp.zeros((12,), jnp.float32)])
    return _edge_call(E, N)(ab_flat, src, dst, ef0, ef1, ef2, wcb)


# speedup vs baseline: 22.3820x; 1.3451x over previous
import functools

import jax
import jax.numpy as jnp
from jax import lax
from jax.experimental import pallas as pl
from jax.experimental.pallas import tpu as pltpu
from jax.experimental.pallas import tpu_sc as plsc

NC = 2
NS = 16
NW = NC * NS
LANES = 16
CH = 80
NPAD = 10240


def _mesh():
    return plsc.VectorSubcoreMesh(core_axis_name="c", subcore_axis_name="s")


def _deg_call(E):
    ept = E // NS
    rows = ept // CH
    rpt = NPAD // NS

    @functools.partial(
        pl.kernel,
        out_type=(jax.ShapeDtypeStruct((NPAD,), jnp.float32),
                  jax.ShapeDtypeStruct((NPAD,), jnp.float32)),
        mesh=_mesh(),
        scratch_types=[
            pltpu.VMEM_SHARED((NPAD,), jnp.float32),
            pltpu.VMEM((ept,), jnp.int32),
            pltpu.VMEM((CH,), jnp.float32),
        ],
    )
    def deg_kernel(src_h, dst_h, ones_h, zeros_h, out_o, out_i,
                   deg_sh, idx_v, ones_v):
        c = lax.axis_index("c")
        s = lax.axis_index("s")
        z0 = s * rpt
        pltpu.sync_copy(zeros_h.at[pl.ds(z0, rpt)],
                        deg_sh.at[pl.ds(z0, rpt)])
        pltpu.sync_copy(ones_h, ones_v)

        e0 = s * ept

        @pl.when(c == 0)
        def _():
            pltpu.sync_copy(src_h.at[pl.ds(e0, ept)], idx_v)

        @pl.when(c == 1)
        def _():
            pltpu.sync_copy(dst_h.at[pl.ds(e0, ept)], idx_v)

        plsc.subcore_barrier()

        def step(j, carry):
            o = pl.multiple_of(j * CH, 8)
            pltpu.sync_copy(ones_v, deg_sh.at[idx_v.at[pl.ds(o, CH)]],
                            add=True)
            return carry

        lax.fori_loop(0, rows, step, 0)
        plsc.subcore_barrier()

        @pl.when(c == 0)
        def _():
            pltpu.sync_copy(deg_sh.at[pl.ds(z0, rpt)], out_o.at[pl.ds(z0, rpt)])

        @pl.when(c == 1)
        def _():
            pltpu.sync_copy(deg_sh.at[pl.ds(z0, rpt)], out_i.at[pl.ds(z0, rpt)])

    return deg_kernel


def _agg_call(E, D):
    ept = E // NW
    rows = ept // CH
    rpt = NPAD // NS

    @functools.partial(
        pl.kernel,
        out_type=jax.ShapeDtypeStruct((NC, NPAD, D), jnp.float32),
        mesh=_mesh(),
        scratch_types=[
            pltpu.VMEM_SHARED((NPAD, D), jnp.float32),
            pltpu.VMEM((ept,), jnp.int32),
            pltpu.VMEM((ept,), jnp.int32),
            pltpu.VMEM((CH, D), jnp.float32),
            pltpu.VMEM((CH, D), jnp.float32),
            pltpu.SemaphoreType.DMA,
            pltpu.SemaphoreType.DMA,
            pltpu.SemaphoreType.DMA,
            pltpu.SemaphoreType.DMA,
        ],
    )
    def agg_kernel(table, src_h, dst_h, zeros_h, out,
                   acc_sh, si_v, di_v, buf0, buf1, g0, g1, s0, s1):
        c = lax.axis_index("c")
        s = lax.axis_index("s")
        r0 = s * rpt
        pltpu.sync_copy(zeros_h.at[pl.ds(r0, rpt)], acc_sh.at[pl.ds(r0, rpt)])
        w = c * NS + s
        e0 = w * ept
        pltpu.sync_copy(src_h.at[pl.ds(e0, ept)], si_v)
        pltpu.sync_copy(dst_h.at[pl.ds(e0, ept)], di_v)
        plsc.subcore_barrier()

        def sidx(j):
            return si_v.at[pl.ds(pl.multiple_of(j * CH, 8), CH)]

        def didx(j):
            return di_v.at[pl.ds(pl.multiple_of(j * CH, 8), CH)]

        pltpu.async_copy(table.at[sidx(0)], buf0, g0)

        def step(j, carry):
            def halfstep(bufa, bufb, ga, gb, sa, sb):
                pltpu.make_async_copy(table.at[sidx(j)], bufa, ga).wait()

                @pl.when(j >= 1)
                def _():
                    pltpu.make_async_copy(bufb, acc_sh.at[didx(j - 1)],
                                          sb).wait()

                @pl.when(j + 1 < rows)
                def _():
                    pltpu.async_copy(table.at[sidx(j + 1)], bufb, gb)

                pltpu.async_copy(bufa, acc_sh.at[didx(j)], sa, add=True)

            @pl.when(j % 2 == 0)
            def _():
                halfstep(buf0, buf1, g0, g1, s0, s1)

            @pl.when(j % 2 == 1)
            def _():
                halfstep(buf1, buf0, g1, g0, s1, s0)

            return carry

        lax.fori_loop(0, rows, step, 0)
        if rows % 2 == 1:
            pltpu.make_async_copy(buf0, acc_sh.at[didx(rows - 1)], s0).wait()
        else:
            pltpu.make_async_copy(buf1, acc_sh.at[didx(rows - 1)], s1).wait()
        plsc.subcore_barrier()

        @pl.when(c == 0)
        def _():
            pltpu.sync_copy(acc_sh.at[pl.ds(r0, rpt)], out.at[0, pl.ds(r0, rpt)])

        @pl.when(c == 1)
        def _():
            pltpu.sync_copy(acc_sh.at[pl.ds(r0, rpt)], out.at[1, pl.ds(r0, rpt)])

    return agg_kernel


def _edge_call(E, N):
    ept = E // NW
    blk = 2000
    nb = ept // blk
    nv = blk // LANES

    @functools.partial(
        pl.kernel,
        out_type=jax.ShapeDtypeStruct((E,), jnp.float32),
        mesh=_mesh(),
        scratch_types=[
            pltpu.VMEM((NPAD,), jnp.float32),
            pltpu.VMEM((NPAD,), jnp.float32),
            pltpu.VMEM((blk,), jnp.int32),
            pltpu.VMEM((blk,), jnp.int32),
            pltpu.VMEM((blk,), jnp.float32),
            pltpu.VMEM((blk,), jnp.float32),
            pltpu.VMEM((blk,), jnp.float32),
            pltpu.VMEM((blk,), jnp.float32),
            pltpu.VMEM((LANES,), jnp.float32),
        ],
        compiler_params=pltpu.CompilerParams(needs_layout_passes=False),
    )
    def edge_kernel(a_h, b_h, src, dst, ef0, ef1, ef2, wcb, out,
                    a_v, b_v, s_v, d_v, e0_v, e1_v, e2_v, o_v, w_v):
        c = lax.axis_index("c")
        s = lax.axis_index("s")
        base = (c * NS + s) * ept
        pltpu.sync_copy(a_h, a_v)
        pltpu.sync_copy(b_h, b_v)
        pltpu.sync_copy(wcb, w_v)
        wvec = w_v[...]
        wc0 = wvec[0]
        wc1 = wvec[1]
        wc2 = wvec[2]
        bfc = wvec[3]

        def blk_step(bi, carry):
            off = base + bi * blk
            pltpu.sync_copy(src.at[pl.ds(off, blk)], s_v)
            pltpu.sync_copy(dst.at[pl.ds(off, blk)], d_v)
            pltpu.sync_copy(ef0.at[pl.ds(off, blk)], e0_v)
            pltpu.sync_copy(ef1.at[pl.ds(off, blk)], e1_v)
            pltpu.sync_copy(ef2.at[pl.ds(off, blk)], e2_v)

            def vstep(k, carry2):
                sl = pl.ds(k * LANES, LANES)
                av = plsc.load_gather(a_v, [s_v[sl]])
                bv = plsc.load_gather(b_v, [d_v[sl]])
                x = (av + bv + e0_v[sl] * wc0 + e1_v[sl] * wc1
                     + e2_v[sl] * wc2 + bfc)
                o_v[sl] = 1.0 / (1.0 + jnp.exp(-x))
                return carry2

            lax.fori_loop(0, nv, vstep, 0)
            pltpu.sync_copy(o_v, out.at[pl.ds(off, blk)])
            return carry

        lax.fori_loop(0, nb, blk_step, 0)

    return edge_kernel


def _tc_pre(x, W1, deg_t):
    N, D = x.shape
    H = W1.shape[1]
    R = 1000

    def body(x_ref, w_ref, dg_ref, o_ref):
        ns = lax.rsqrt(jnp.maximum(dg_ref[:, 0:1], 1.0))
        o_ref[...] = jnp.dot(x_ref[...], w_ref[...],
                             preferred_element_type=jnp.float32) * ns

    return pl.pallas_call(
        body,
        grid=(N // R,),
        in_specs=[
            pl.BlockSpec((R, D), lambda i: (i, 0)),
            pl.BlockSpec((D, H), lambda i: (0, 0)),
            pl.BlockSpec((R, 2), lambda i: (i, 0)),
        ],
        out_specs=pl.BlockSpec((R, H), lambda i: (i, 0)),
        out_shape=jax.ShapeDtypeStruct((N, H), jnp.float32),
    )(x, W1, deg_t)


def _tc_mid(acc, deg_t, b1, W2, Wab):
    N = deg_t.shape[0]
    H = W2.shape[0]
    R = 1000

    def body(a_ref, dg_ref, b_ref, w_ref, wab_ref, o_ref):
        asum = a_ref[0] + a_ref[1]
        nd = lax.rsqrt(jnp.maximum(dg_ref[:, 1:2], 1.0))
        h1 = jnp.maximum(asum * nd + b_ref[...], 0.0)
        ns = lax.rsqrt(jnp.maximum(dg_ref[:, 0:1], 1.0))
        xs2 = jnp.dot(h1, w_ref[...],
                      preferred_element_type=jnp.float32) * ns
        o_ref[...] = jnp.dot(xs2, wab_ref[...],
                             preferred_element_type=jnp.float32)

    return pl.pallas_call(
        body,
        grid=(N // R,),
        in_specs=[
            pl.BlockSpec((2, R, H), lambda i: (0, i, 0)),
            pl.BlockSpec((R, 2), lambda i: (i, 0)),
            pl.BlockSpec((1, H), lambda i: (0, 0)),
            pl.BlockSpec((H, H), lambda i: (0, 0)),
            pl.BlockSpec((H, 2), lambda i: (0, 0)),
        ],
        out_specs=pl.BlockSpec((R, 2), lambda i: (i, 0)),
        out_shape=jax.ShapeDtypeStruct((N, 2), jnp.float32),
    )(acc, deg_t, b1, W2, Wab)


def _agg2_call(E, N):
    ept = E // NW
    rows = ept // CH
    vs = CH // LANES
    rpt = NPAD // NS

    @functools.partial(
        pl.kernel,
        out_type=jax.ShapeDtypeStruct((4 * NPAD,), jnp.float32),
        mesh=_mesh(),
        scratch_types=[
            pltpu.VMEM_SHARED((NPAD,), jnp.float32),
            pltpu.VMEM_SHARED((NPAD,), jnp.float32),
            pltpu.VMEM((N,), jnp.float32),
            pltpu.VMEM((N,), jnp.float32),
            pltpu.VMEM((ept,), jnp.int32),
            pltpu.VMEM((ept,), jnp.int32),
            pltpu.VMEM((CH,), jnp.float32),
            pltpu.VMEM((CH,), jnp.float32),
        ],
        compiler_params=pltpu.CompilerParams(needs_layout_passes=False),
    )
    def agg2_kernel(t_h, u_h, src_h, dst_h, zeros_h, out,
                    t_sh, u_sh, t_v, u_v, si_v, di_v, tp_v, up_v):
        c = lax.axis_index("c")
        s = lax.axis_index("s")
        z0 = s * rpt
        pltpu.sync_copy(zeros_h.at[pl.ds(z0, rpt)], t_sh.at[pl.ds(z0, rpt)])
        pltpu.sync_copy(zeros_h.at[pl.ds(z0, rpt)], u_sh.at[pl.ds(z0, rpt)])
        pltpu.sync_copy(t_h, t_v)
        pltpu.sync_copy(u_h, u_v)
        e0 = (c * NS + s) * ept
        pltpu.sync_copy(src_h.at[pl.ds(e0, ept)], si_v)
        pltpu.sync_copy(dst_h.at[pl.ds(e0, ept)], di_v)
        plsc.subcore_barrier()

        def step(j, carry):
            o = pl.multiple_of(j * CH, 8)
            for k in range(vs):
                sl = pl.ds(o + k * LANES, LANES)
                sv = si_v[sl]
                tp_v[pl.ds(k * LANES, LANES)] = plsc.load_gather(t_v, [sv])
                up_v[pl.ds(k * LANES, LANES)] = plsc.load_gather(u_v, [sv])
            didx = di_v.at[pl.ds(o, CH)]
            pltpu.sync_copy(tp_v, t_sh.at[didx], add=True)
            pltpu.sync_copy(up_v, u_sh.at[didx], add=True)
            return carry

        lax.fori_loop(0, rows, step, 0)
        plsc.subcore_barrier()
        base = pl.multiple_of(c * (2 * NPAD) + z0, 8)
        pltpu.sync_copy(t_sh.at[pl.ds(z0, rpt)], out.at[pl.ds(base, rpt)])
        pltpu.sync_copy(u_sh.at[pl.ds(z0, rpt)],
                        out.at[pl.ds(base + NPAD, rpt)])

    return agg2_kernel


def _tc_post2(t0, t1, u0, u1, dgi, b2, Wab):
    H = Wab.shape[0]
    R = 2048
    NP = dgi.shape[0]

    def body(t0_ref, t1_ref, u0_ref, u1_ref, dg_ref, b_ref, w_ref,
             a_ref, bv_ref):
        cab = jnp.dot(b_ref[...], w_ref[...],
                      preferred_element_type=jnp.float32)
        nd = lax.rsqrt(jnp.maximum(dg_ref[...], 1.0))
        a_ref[...] = (t0_ref[...] + t1_ref[...]) * nd + cab[0, 0]
        bv_ref[...] = (u0_ref[...] + u1_ref[...]) * nd + cab[0, 1]

    vec = pl.BlockSpec((R,), lambda i: (i,))
    return pl.pallas_call(
        body,
        grid=(NP // R,),
        in_specs=[
            vec, vec, vec, vec, vec,
            pl.BlockSpec((1, H), lambda i: (0, 0)),
            pl.BlockSpec((H, 2), lambda i: (0, 0)),
        ],
        out_specs=(vec, vec),
        out_shape=(jax.ShapeDtypeStruct((NP,), jnp.float32),
                   jax.ShapeDtypeStruct((NP,), jnp.float32)),
    )(t0, t1, u0, u1, dgi, b2, Wab)


def kernel(node_features, edge_index, edge_features, W1, b1, W2, b2, Wfc, bfc):
    N, D = node_features.shape
    E = edge_index.shape[1]
    H = W1.shape[1]

    src = edge_index[0]
    dst = edge_index[1]
    eft = edge_features.T
    ef0 = eft[0]
    ef1 = eft[1]
    ef2 = eft[2]

    zeros_nd = jnp.zeros((NPAD, H), jnp.float32)
    zeros_p = jnp.zeros((NPAD,), jnp.float32)
    ones_c = jnp.ones((CH,), jnp.float32)

    deg_o, deg_i = _deg_call(E)(src, dst, ones_c, zeros_p)
    deg_t = jnp.stack([deg_o[:N], deg_i[:N]], axis=1)

    xs1 = _tc_pre(node_features, W1, deg_t)
    acc1 = _agg_call(E, H)(xs1, src, dst, zeros_nd)

    Wab = jnp.concatenate([Wfc[:H], Wfc[H:2 * H]], axis=1)
    tu = _tc_mid(acc1, deg_t, b1.reshape(1, H), W2, Wab)
    t = tu[:, 0]
    u = tu[:, 1]
    out4 = _agg2_call(E, N)(t, u, src, dst, zeros_p)
    a_n, b_n = _tc_post2(out4[:NPAD], out4[2 * NPAD:3 * NPAD],
                         out4[NPAD:2 * NPAD], out4[3 * NPAD:],
                         deg_i, b2.reshape(1, H), Wab)

    wcb = jnp.concatenate(
        [Wfc[2 * H:, 0], bfc, jnp.zeros((12,), jnp.float32)])
    return _edge_call(E, N)(a_n, b_n, src, dst, ef0, ef1, ef2, wcb)
